# Initial kernel scaffold; baseline (speedup 1.0000x reference)
import jax
import jax.numpy as jnp
from jax.experimental import pallas as pl


def kernel(x, edge_index, W_rel1, b_rel1, W_root1, W_rel2, b_rel2, W_root2, W_rel3, b_rel3, W_root3, W_lin, b_lin):
    raise NotImplementedError("write your pallas kernel here")


# speedup vs baseline: 12.8687x; 12.8687x over previous
import functools

import jax
import jax.numpy as jnp
from jax import lax
from jax.experimental import pallas as pl
from jax.experimental.pallas import tpu as pltpu
from jax.experimental.pallas import tpu_sc as plsc

N = 10000
E = 320000
F = 128
H = 20
L = 16

NP = 10240
FP = 32
NC = 2
NS = 16
NW = NC * NS
BB = 128
NB = (E + NW * BB - 1) // (NW * BB)
EP = NW * BB * ((E + NW * BB - 1) // (NW * BB))
NB = EP // (NW * BB)
RPT = NP // NS

_mesh = plsc.VectorSubcoreMesh(core_axis_name="c", subcore_axis_name="s")


@functools.partial(
    pl.kernel,
    out_type=jax.ShapeDtypeStruct((NC, NP, FP), jnp.float32),
    mesh=_mesh,
    compiler_params=pltpu.CompilerParams(use_tc_tiling_on_sc=False),
    scratch_types=[
        pltpu.VMEM((NB, BB), jnp.int32),
        pltpu.VMEM((NB, BB), jnp.int32),
        pltpu.VMEM((2, BB, FP), jnp.float32),
        pltpu.VMEM_SHARED((NP, FP), jnp.float32),
        pltpu.SemaphoreType.DMA,
    ],
)
def _sc_segment_sum(y_hbm, src_hbm, dst_hbm, zeros_hbm, out_hbm,
                    src_v, dst_v, rows_v, acc, gsem):
    c = lax.axis_index("c")
    s = lax.axis_index("s")
    wid = s * NC + c

    pltpu.sync_copy(zeros_hbm, acc.at[pl.ds(s * RPT, RPT)])
    pltpu.sync_copy(src_hbm.at[wid], src_v)
    pltpu.sync_copy(dst_hbm.at[wid], dst_v)
    plsc.subcore_barrier()

    pltpu.async_copy(y_hbm.at[src_v.at[0]], rows_v.at[0], gsem)

    def body(j, carry):
        buf = lax.rem(j, 2)

        @pl.when(j + 1 < NB)
        def _():
            pltpu.async_copy(y_hbm.at[src_v.at[j + 1]], rows_v.at[1 - buf], gsem)

        pltpu.make_async_copy(y_hbm.at[src_v.at[j]], rows_v.at[buf], gsem).wait()
        pltpu.sync_copy(rows_v.at[buf], acc.at[dst_v.at[j]], add=True)
        return carry

    lax.fori_loop(0, NB, body, 0)

    plsc.subcore_barrier()
    pltpu.sync_copy(acc.at[pl.ds(s * RPT, RPT)],
                    out_hbm.at[c, pl.ds(s * RPT, RPT)])


BM = 1024


def _mm2_body(x_ref, wa_ref, wb_ref, ya_ref, yb_ref):
    xv = x_ref[...]
    ya_ref[...] = jnp.dot(xv, wa_ref[...], preferred_element_type=jnp.float32)
    yb_ref[...] = jnp.dot(xv, wb_ref[...], preferred_element_type=jnp.float32)


def _tc_mm2(xp, wa, wb):
    k = xp.shape[1]
    return pl.pallas_call(
        _mm2_body,
        grid=(NP // BM,),
        in_specs=[
            pl.BlockSpec((BM, k), lambda i: (i, 0)),
            pl.BlockSpec((k, FP), lambda i: (0, 0)),
            pl.BlockSpec((k, FP), lambda i: (0, 0)),
        ],
        out_specs=[
            pl.BlockSpec((BM, FP), lambda i: (i, 0)),
            pl.BlockSpec((BM, FP), lambda i: (i, 0)),
        ],
        out_shape=[
            jax.ShapeDtypeStruct((NP, FP), jnp.float32),
            jax.ShapeDtypeStruct((NP, FP), jnp.float32),
        ],
    )(xp, wa, wb)


def _layer_body(s_ref, r_ref, b_ref, wa_ref, wb_ref, emb_ref, ya_ref, yb_ref):
    e = jnp.maximum(s_ref[0] + s_ref[1] + r_ref[...] + b_ref[...], 0.0)
    emb_ref[...] = e
    ya_ref[...] = jnp.dot(e, wa_ref[...], preferred_element_type=jnp.float32)
    yb_ref[...] = jnp.dot(e, wb_ref[...], preferred_element_type=jnp.float32)


def _tc_layer(sparts, r, b, wa, wb):
    return pl.pallas_call(
        _layer_body,
        grid=(NP // BM,),
        in_specs=[
            pl.BlockSpec((NC, BM, FP), lambda i: (0, i, 0)),
            pl.BlockSpec((BM, FP), lambda i: (i, 0)),
            pl.BlockSpec((1, FP), lambda i: (0, 0)),
            pl.BlockSpec((FP, FP), lambda i: (0, 0)),
            pl.BlockSpec((FP, FP), lambda i: (0, 0)),
        ],
        out_specs=[
            pl.BlockSpec((BM, FP), lambda i: (i, 0)),
            pl.BlockSpec((BM, FP), lambda i: (i, 0)),
            pl.BlockSpec((BM, FP), lambda i: (i, 0)),
        ],
        out_shape=[
            jax.ShapeDtypeStruct((NP, FP), jnp.float32),
            jax.ShapeDtypeStruct((NP, FP), jnp.float32),
            jax.ShapeDtypeStruct((NP, FP), jnp.float32),
        ],
    )(sparts, r, b, wa, wb)


def _final_body(s_ref, r_ref, b_ref, e1_ref, e2_ref, w1_ref, w2_ref, w3_ref,
                bl_ref, out_ref):
    e3 = jnp.maximum(s_ref[0] + s_ref[1] + r_ref[...] + b_ref[...], 0.0)
    acc = jnp.dot(e1_ref[...], w1_ref[...], preferred_element_type=jnp.float32)
    acc += jnp.dot(e2_ref[...], w2_ref[...], preferred_element_type=jnp.float32)
    acc += jnp.dot(e3, w3_ref[...], preferred_element_type=jnp.float32)
    out_ref[...] = acc + bl_ref[...]


def _tc_final(sparts, r, b, e1, e2, w1, w2, w3, bl):
    return pl.pallas_call(
        _final_body,
        grid=(NP // BM,),
        in_specs=[
            pl.BlockSpec((NC, BM, FP), lambda i: (0, i, 0)),
            pl.BlockSpec((BM, FP), lambda i: (i, 0)),
            pl.BlockSpec((1, FP), lambda i: (0, 0)),
            pl.BlockSpec((BM, FP), lambda i: (i, 0)),
            pl.BlockSpec((BM, FP), lambda i: (i, 0)),
            pl.BlockSpec((FP, L), lambda i: (0, 0)),
            pl.BlockSpec((FP, L), lambda i: (0, 0)),
            pl.BlockSpec((FP, L), lambda i: (0, 0)),
            pl.BlockSpec((1, L), lambda i: (0, 0)),
        ],
        out_specs=pl.BlockSpec((BM, L), lambda i: (i, 0)),
        out_shape=jax.ShapeDtypeStruct((NP, L), jnp.float32),
    )(sparts, r, b, e1, e2, w1, w2, w3, bl)


def _padw(wt, rows, cols):
    out = jnp.zeros((rows, cols), jnp.float32)
    return out.at[: wt.shape[0], : wt.shape[1]].set(wt)


def kernel(x, edge_index, W_rel1, b_rel1, W_root1, W_rel2, b_rel2, W_root2,
           W_rel3, b_rel3, W_root3, W_lin, b_lin):
    src = edge_index[0].astype(jnp.int32)
    dst = edge_index[1].astype(jnp.int32)
    pad = EP - E
    srcp = jnp.concatenate([src, jnp.zeros((pad,), jnp.int32)]).reshape(NW, NB, BB)
    dstp = jnp.concatenate([dst, jnp.full((pad,), NP - 1, jnp.int32)]).reshape(NW, NB, BB)
    xp = jnp.pad(x, ((0, NP - N), (0, 0)))
    zeros_tile = jnp.zeros((RPT, FP), jnp.float32)

    wr1 = _padw(W_rel1.T, F, FP)
    wt1 = _padw(W_root1.T, F, FP)
    wr2 = _padw(W_rel2.T, FP, FP)
    wt2 = _padw(W_root2.T, FP, FP)
    wr3 = _padw(W_rel3.T, FP, FP)
    wt3 = _padw(W_root3.T, FP, FP)
    b1 = _padw(b_rel1[None, :], 1, FP)
    b2 = _padw(b_rel2[None, :], 1, FP)
    b3 = _padw(b_rel3[None, :], 1, FP)
    wl1 = _padw(W_lin[:, 0 * H:1 * H].T, FP, L)
    wl2 = _padw(W_lin[:, 1 * H:2 * H].T, FP, L)
    wl3 = _padw(W_lin[:, 2 * H:3 * H].T, FP, L)
    bl = b_lin[None, :]

    y1, r1 = _tc_mm2(xp, wr1, wt1)
    s1 = _sc_segment_sum(y1, srcp, dstp, zeros_tile)
    emb1, y2, r2 = _tc_layer(s1, r1, b1, wr2, wt2)
    s2 = _sc_segment_sum(y2, srcp, dstp, zeros_tile)
    emb2, y3, r3 = _tc_layer(s2, r2, b2, wr3, wt3)
    s3 = _sc_segment_sum(y3, srcp, dstp, zeros_tile)
    out = _tc_final(s3, r3, b3, emb1, emb2, wl1, wl2, wl3, bl)
    return out[:N]


# speedup vs baseline: 13.6752x; 1.0627x over previous
import functools

import jax
import jax.numpy as jnp
from jax import lax
from jax.experimental import pallas as pl
from jax.experimental.pallas import tpu as pltpu
from jax.experimental.pallas import tpu_sc as plsc

N = 10000
E = 320000
F = 128
H = 20
L = 16

NP = 10240
FP = 32
NC = 2
NS = 16
NW = NC * NS
BB = 128
NB = (E + NW * BB - 1) // (NW * BB)
EP = NW * BB * ((E + NW * BB - 1) // (NW * BB))
NB = EP // (NW * BB)
RPT = NP // NS
NBUF = 8
PLA = 4
DLAG = 4

_mesh = plsc.VectorSubcoreMesh(core_axis_name="c", subcore_axis_name="s")


@functools.partial(
    pl.kernel,
    out_type=jax.ShapeDtypeStruct((NC, NP, FP), jnp.float32),
    mesh=_mesh,
    compiler_params=pltpu.CompilerParams(use_tc_tiling_on_sc=False),
    scratch_types=[
        pltpu.VMEM((NB, BB), jnp.int32),
        pltpu.VMEM((NB, BB), jnp.int32),
        pltpu.VMEM((NBUF, BB, FP), jnp.float32),
        pltpu.VMEM_SHARED((NP, FP), jnp.float32),
        pltpu.SemaphoreType.DMA,
        pltpu.SemaphoreType.DMA,
    ],
)
def _sc_segment_sum(y_hbm, src_hbm, dst_hbm, zeros_hbm, out_hbm,
                    src_v, dst_v, rows_v, acc, gsem, ssem):
    c = lax.axis_index("c")
    s = lax.axis_index("s")
    wid = s * NC + c

    pltpu.sync_copy(zeros_hbm, acc.at[pl.ds(s * RPT, RPT)])
    pltpu.sync_copy(src_hbm.at[wid], src_v)
    pltpu.sync_copy(dst_hbm.at[wid], dst_v)
    plsc.subcore_barrier()

    for b in range(PLA):
        pltpu.async_copy(y_hbm.at[src_v.at[b]], rows_v.at[b], gsem)

    def body(j, carry):
        buf = lax.rem(j, NBUF)
        pltpu.make_async_copy(y_hbm.at[src_v.at[j]], rows_v.at[buf], gsem).wait()
        pltpu.async_copy(rows_v.at[buf], acc.at[dst_v.at[j]], ssem, add=True)

        @pl.when(j >= DLAG)
        def _drain():
            ob = lax.rem(j - DLAG, NBUF)
            pltpu.make_async_copy(rows_v.at[ob], acc.at[dst_v.at[j - DLAG]],
                                  ssem).wait()

        @pl.when(j + PLA < NB)
        def _prefetch():
            gb = lax.rem(j + PLA, NBUF)
            pltpu.async_copy(y_hbm.at[src_v.at[j + PLA]], rows_v.at[gb], gsem)

        return carry

    lax.fori_loop(0, NB, body, 0)
    for t in range(DLAG):
        j = NB - DLAG + t
        pltpu.make_async_copy(rows_v.at[j % NBUF], acc.at[dst_v.at[j]],
                              ssem).wait()

    plsc.subcore_barrier()
    pltpu.sync_copy(acc.at[pl.ds(s * RPT, RPT)],
                    out_hbm.at[c, pl.ds(s * RPT, RPT)])


BM = 1024


def _mm2_body(x_ref, wa_ref, wb_ref, ya_ref, yb_ref):
    xv = x_ref[...]
    ya_ref[...] = jnp.dot(xv, wa_ref[...], preferred_element_type=jnp.float32)
    yb_ref[...] = jnp.dot(xv, wb_ref[...], preferred_element_type=jnp.float32)


def _tc_mm2(xp, wa, wb):
    k = xp.shape[1]
    return pl.pallas_call(
        _mm2_body,
        grid=(NP // BM,),
        in_specs=[
            pl.BlockSpec((BM, k), lambda i: (i, 0)),
            pl.BlockSpec((k, FP), lambda i: (0, 0)),
            pl.BlockSpec((k, FP), lambda i: (0, 0)),
        ],
        out_specs=[
            pl.BlockSpec((BM, FP), lambda i: (i, 0)),
            pl.BlockSpec((BM, FP), lambda i: (i, 0)),
        ],
        out_shape=[
            jax.ShapeDtypeStruct((NP, FP), jnp.float32),
            jax.ShapeDtypeStruct((NP, FP), jnp.float32),
        ],
    )(xp, wa, wb)


def _layer_body(s_ref, r_ref, b_ref, wa_ref, wb_ref, emb_ref, ya_ref, yb_ref):
    e = jnp.maximum(s_ref[0] + s_ref[1] + r_ref[...] + b_ref[...], 0.0)
    emb_ref[...] = e
    ya_ref[...] = jnp.dot(e, wa_ref[...], preferred_element_type=jnp.float32)
    yb_ref[...] = jnp.dot(e, wb_ref[...], preferred_element_type=jnp.float32)


def _tc_layer(sparts, r, b, wa, wb):
    return pl.pallas_call(
        _layer_body,
        grid=(NP // BM,),
        in_specs=[
            pl.BlockSpec((NC, BM, FP), lambda i: (0, i, 0)),
            pl.BlockSpec((BM, FP), lambda i: (i, 0)),
            pl.BlockSpec((1, FP), lambda i: (0, 0)),
            pl.BlockSpec((FP, FP), lambda i: (0, 0)),
            pl.BlockSpec((FP, FP), lambda i: (0, 0)),
        ],
        out_specs=[
            pl.BlockSpec((BM, FP), lambda i: (i, 0)),
            pl.BlockSpec((BM, FP), lambda i: (i, 0)),
            pl.BlockSpec((BM, FP), lambda i: (i, 0)),
        ],
        out_shape=[
            jax.ShapeDtypeStruct((NP, FP), jnp.float32),
            jax.ShapeDtypeStruct((NP, FP), jnp.float32),
            jax.ShapeDtypeStruct((NP, FP), jnp.float32),
        ],
    )(sparts, r, b, wa, wb)


def _final_body(s_ref, r_ref, b_ref, e1_ref, e2_ref, w1_ref, w2_ref, w3_ref,
                bl_ref, out_ref):
    e3 = jnp.maximum(s_ref[0] + s_ref[1] + r_ref[...] + b_ref[...], 0.0)
    acc = jnp.dot(e1_ref[...], w1_ref[...], preferred_element_type=jnp.float32)
    acc += jnp.dot(e2_ref[...], w2_ref[...], preferred_element_type=jnp.float32)
    acc += jnp.dot(e3, w3_ref[...], preferred_element_type=jnp.float32)
    out_ref[...] = acc + bl_ref[...]


def _tc_final(sparts, r, b, e1, e2, w1, w2, w3, bl):
    return pl.pallas_call(
        _final_body,
        grid=(NP // BM,),
        in_specs=[
            pl.BlockSpec((NC, BM, FP), lambda i: (0, i, 0)),
            pl.BlockSpec((BM, FP), lambda i: (i, 0)),
            pl.BlockSpec((1, FP), lambda i: (0, 0)),
            pl.BlockSpec((BM, FP), lambda i: (i, 0)),
            pl.BlockSpec((BM, FP), lambda i: (i, 0)),
            pl.BlockSpec((FP, L), lambda i: (0, 0)),
            pl.BlockSpec((FP, L), lambda i: (0, 0)),
            pl.BlockSpec((FP, L), lambda i: (0, 0)),
            pl.BlockSpec((1, L), lambda i: (0, 0)),
        ],
        out_specs=pl.BlockSpec((BM, L), lambda i: (i, 0)),
        out_shape=jax.ShapeDtypeStruct((NP, L), jnp.float32),
    )(sparts, r, b, e1, e2, w1, w2, w3, bl)


def _padw(wt, rows, cols):
    out = jnp.zeros((rows, cols), jnp.float32)
    return out.at[: wt.shape[0], : wt.shape[1]].set(wt)


def kernel(x, edge_index, W_rel1, b_rel1, W_root1, W_rel2, b_rel2, W_root2,
           W_rel3, b_rel3, W_root3, W_lin, b_lin):
    src = edge_index[0].astype(jnp.int32)
    dst = edge_index[1].astype(jnp.int32)
    pad = EP - E
    srcp = jnp.concatenate([src, jnp.zeros((pad,), jnp.int32)]).reshape(NW, NB, BB)
    dstp = jnp.concatenate([dst, jnp.full((pad,), NP - 1, jnp.int32)]).reshape(NW, NB, BB)
    xp = jnp.pad(x, ((0, NP - N), (0, 0)))
    zeros_tile = jnp.zeros((RPT, FP), jnp.float32)

    wr1 = _padw(W_rel1.T, F, FP)
    wt1 = _padw(W_root1.T, F, FP)
    wr2 = _padw(W_rel2.T, FP, FP)
    wt2 = _padw(W_root2.T, FP, FP)
    wr3 = _padw(W_rel3.T, FP, FP)
    wt3 = _padw(W_root3.T, FP, FP)
    b1 = _padw(b_rel1[None, :], 1, FP)
    b2 = _padw(b_rel2[None, :], 1, FP)
    b3 = _padw(b_rel3[None, :], 1, FP)
    wl1 = _padw(W_lin[:, 0 * H:1 * H].T, FP, L)
    wl2 = _padw(W_lin[:, 1 * H:2 * H].T, FP, L)
    wl3 = _padw(W_lin[:, 2 * H:3 * H].T, FP, L)
    bl = b_lin[None, :]

    y1, r1 = _tc_mm2(xp, wr1, wt1)
    s1 = _sc_segment_sum(y1, srcp, dstp, zeros_tile)
    emb1, y2, r2 = _tc_layer(s1, r1, b1, wr2, wt2)
    s2 = _sc_segment_sum(y2, srcp, dstp, zeros_tile)
    emb2, y3, r3 = _tc_layer(s2, r2, b2, wr3, wt3)
    s3 = _sc_segment_sum(y3, srcp, dstp, zeros_tile)
    out = _tc_final(s3, r3, b3, emb1, emb2, wl1, wl2, wl3, bl)
    return out[:N]


# speedup vs baseline: 16.7273x; 1.2232x over previous
import functools

import jax
import jax.numpy as jnp
from jax import lax
from jax.experimental import pallas as pl
from jax.experimental.pallas import tpu as pltpu
from jax.experimental.pallas import tpu_sc as plsc

N = 10000
E = 320000
F = 128
H = 20
L = 16

NP = 10240
FP = 32
NC = 2
NS = 16
NW = NC * NS
BB = 128
TOTB = (E + BB - 1) // BB
NB0 = 50
NB1 = 107
TOTB_PAD = NS * (NB0 + NB1)
NBMAX = max(NB0, NB1)
EP = (TOTB_PAD + NBMAX) * BB
RPT = NP // NS
NBUF = 8
PLA = 4
DLAG = 4

_mesh = plsc.VectorSubcoreMesh(core_axis_name="c", subcore_axis_name="s")


@functools.partial(
    pl.kernel,
    out_type=jax.ShapeDtypeStruct((NC, NP, FP), jnp.float32),
    mesh=_mesh,
    compiler_params=pltpu.CompilerParams(use_tc_tiling_on_sc=False),
    scratch_types=[
        pltpu.VMEM((NBMAX, BB), jnp.int32),
        pltpu.VMEM((NBMAX, BB), jnp.int32),
        pltpu.VMEM((NBUF, BB, FP), jnp.float32),
        pltpu.VMEM_SHARED((NP, FP), jnp.float32),
        pltpu.SemaphoreType.DMA,
        pltpu.SemaphoreType.DMA,
    ],
)
def _sc_segment_sum(y_hbm, src_hbm, dst_hbm, zeros_hbm, out_hbm,
                    src_v, dst_v, rows_v, acc, gsem, ssem):
    c = lax.axis_index("c")
    s = lax.axis_index("s")
    nb = lax.select(c == 0, NB0, NB1)
    base = c * (NS * NB0) + s * nb

    pltpu.sync_copy(zeros_hbm, acc.at[pl.ds(s * RPT, RPT)])
    pltpu.sync_copy(src_hbm.at[pl.ds(base, NBMAX)], src_v)
    pltpu.sync_copy(dst_hbm.at[pl.ds(base, NBMAX)], dst_v)
    plsc.subcore_barrier()

    for b in range(PLA):
        pltpu.async_copy(y_hbm.at[src_v.at[b]], rows_v.at[b], gsem)

    def body(j, carry):
        buf = lax.rem(j, NBUF)
        pltpu.make_async_copy(y_hbm.at[src_v.at[j]], rows_v.at[buf], gsem).wait()
        pltpu.async_copy(rows_v.at[buf], acc.at[dst_v.at[j]], ssem, add=True)

        @pl.when(j >= DLAG)
        def _drain():
            ob = lax.rem(j - DLAG, NBUF)
            pltpu.make_async_copy(rows_v.at[ob], acc.at[dst_v.at[j - DLAG]],
                                  ssem).wait()

        @pl.when(j + PLA < nb)
        def _prefetch():
            gb = lax.rem(j + PLA, NBUF)
            pltpu.async_copy(y_hbm.at[src_v.at[j + PLA]], rows_v.at[gb], gsem)

        return carry

    lax.fori_loop(0, nb, body, 0)

    def drain_body(j, carry):
        pltpu.make_async_copy(rows_v.at[lax.rem(j, NBUF)],
                              acc.at[dst_v.at[j]], ssem).wait()
        return carry

    lax.fori_loop(nb - DLAG, nb, drain_body, 0)

    plsc.subcore_barrier()
    pltpu.sync_copy(acc.at[pl.ds(s * RPT, RPT)],
                    out_hbm.at[c, pl.ds(s * RPT, RPT)])


BM = 1024


def _mm2_body(x_ref, wa_ref, wb_ref, ya_ref, yb_ref):
    xv = x_ref[...]
    ya_ref[...] = jnp.dot(xv, wa_ref[...], preferred_element_type=jnp.float32)
    yb_ref[...] = jnp.dot(xv, wb_ref[...], preferred_element_type=jnp.float32)


def _tc_mm2(xp, wa, wb):
    k = xp.shape[1]
    return pl.pallas_call(
        _mm2_body,
        grid=(NP // BM,),
        in_specs=[
            pl.BlockSpec((BM, k), lambda i: (i, 0)),
            pl.BlockSpec((k, FP), lambda i: (0, 0)),
            pl.BlockSpec((k, FP), lambda i: (0, 0)),
        ],
        out_specs=[
            pl.BlockSpec((BM, FP), lambda i: (i, 0)),
            pl.BlockSpec((BM, FP), lambda i: (i, 0)),
        ],
        out_shape=[
            jax.ShapeDtypeStruct((NP, FP), jnp.float32),
            jax.ShapeDtypeStruct((NP, FP), jnp.float32),
        ],
    )(xp, wa, wb)


def _layer_body(s_ref, r_ref, b_ref, wa_ref, wb_ref, emb_ref, ya_ref, yb_ref):
    e = jnp.maximum(s_ref[0] + s_ref[1] + r_ref[...] + b_ref[...], 0.0)
    emb_ref[...] = e
    ya_ref[...] = jnp.dot(e, wa_ref[...], preferred_element_type=jnp.float32)
    yb_ref[...] = jnp.dot(e, wb_ref[...], preferred_element_type=jnp.float32)


def _tc_layer(sparts, r, b, wa, wb):
    return pl.pallas_call(
        _layer_body,
        grid=(NP // BM,),
        in_specs=[
            pl.BlockSpec((NC, BM, FP), lambda i: (0, i, 0)),
            pl.BlockSpec((BM, FP), lambda i: (i, 0)),
            pl.BlockSpec((1, FP), lambda i: (0, 0)),
            pl.BlockSpec((FP, FP), lambda i: (0, 0)),
            pl.BlockSpec((FP, FP), lambda i: (0, 0)),
        ],
        out_specs=[
            pl.BlockSpec((BM, FP), lambda i: (i, 0)),
            pl.BlockSpec((BM, FP), lambda i: (i, 0)),
            pl.BlockSpec((BM, FP), lambda i: (i, 0)),
        ],
        out_shape=[
            jax.ShapeDtypeStruct((NP, FP), jnp.float32),
            jax.ShapeDtypeStruct((NP, FP), jnp.float32),
            jax.ShapeDtypeStruct((NP, FP), jnp.float32),
        ],
    )(sparts, r, b, wa, wb)


def _final_body(s_ref, r_ref, b_ref, e1_ref, e2_ref, w1_ref, w2_ref, w3_ref,
                bl_ref, out_ref):
    e3 = jnp.maximum(s_ref[0] + s_ref[1] + r_ref[...] + b_ref[...], 0.0)
    acc = jnp.dot(e1_ref[...], w1_ref[...], preferred_element_type=jnp.float32)
    acc += jnp.dot(e2_ref[...], w2_ref[...], preferred_element_type=jnp.float32)
    acc += jnp.dot(e3, w3_ref[...], preferred_element_type=jnp.float32)
    out_ref[...] = acc + bl_ref[...]


def _tc_final(sparts, r, b, e1, e2, w1, w2, w3, bl):
    return pl.pallas_call(
        _final_body,
        grid=(NP // BM,),
        in_specs=[
            pl.BlockSpec((NC, BM, FP), lambda i: (0, i, 0)),
            pl.BlockSpec((BM, FP), lambda i: (i, 0)),
            pl.BlockSpec((1, FP), lambda i: (0, 0)),
            pl.BlockSpec((BM, FP), lambda i: (i, 0)),
            pl.BlockSpec((BM, FP), lambda i: (i, 0)),
            pl.BlockSpec((FP, L), lambda i: (0, 0)),
            pl.BlockSpec((FP, L), lambda i: (0, 0)),
            pl.BlockSpec((FP, L), lambda i: (0, 0)),
            pl.BlockSpec((1, L), lambda i: (0, 0)),
        ],
        out_specs=pl.BlockSpec((BM, L), lambda i: (i, 0)),
        out_shape=jax.ShapeDtypeStruct((NP, L), jnp.float32),
    )(sparts, r, b, e1, e2, w1, w2, w3, bl)


def _padw(wt, rows, cols):
    out = jnp.zeros((rows, cols), jnp.float32)
    return out.at[: wt.shape[0], : wt.shape[1]].set(wt)


def kernel(x, edge_index, W_rel1, b_rel1, W_root1, W_rel2, b_rel2, W_root2,
           W_rel3, b_rel3, W_root3, W_lin, b_lin):
    src = edge_index[0].astype(jnp.int32)
    dst = edge_index[1].astype(jnp.int32)
    pad = EP - E
    srcp = jnp.concatenate([src, jnp.zeros((pad,), jnp.int32)]).reshape(-1, BB)
    dstp = jnp.concatenate([dst, jnp.full((pad,), NP - 1, jnp.int32)]).reshape(-1, BB)
    xp = jnp.pad(x, ((0, NP - N), (0, 0)))
    zeros_tile = jnp.zeros((RPT, FP), jnp.float32)

    wr1 = _padw(W_rel1.T, F, FP)
    wt1 = _padw(W_root1.T, F, FP)
    wr2 = _padw(W_rel2.T, FP, FP)
    wt2 = _padw(W_root2.T, FP, FP)
    wr3 = _padw(W_rel3.T, FP, FP)
    wt3 = _padw(W_root3.T, FP, FP)
    b1 = _padw(b_rel1[None, :], 1, FP)
    b2 = _padw(b_rel2[None, :], 1, FP)
    b3 = _padw(b_rel3[None, :], 1, FP)
    wl1 = _padw(W_lin[:, 0 * H:1 * H].T, FP, L)
    wl2 = _padw(W_lin[:, 1 * H:2 * H].T, FP, L)
    wl3 = _padw(W_lin[:, 2 * H:3 * H].T, FP, L)
    bl = b_lin[None, :]

    y1, r1 = _tc_mm2(xp, wr1, wt1)
    s1 = _sc_segment_sum(y1, srcp, dstp, zeros_tile)
    emb1, y2, r2 = _tc_layer(s1, r1, b1, wr2, wt2)
    s2 = _sc_segment_sum(y2, srcp, dstp, zeros_tile)
    emb2, y3, r3 = _tc_layer(s2, r2, b2, wr3, wt3)
    s3 = _sc_segment_sum(y3, srcp, dstp, zeros_tile)
    out = _tc_final(s3, r3, b3, emb1, emb2, wl1, wl2, wl3, bl)
    return out[:N]


# speedup vs baseline: 18.2176x; 1.0891x over previous
import functools

import jax
import jax.numpy as jnp
from jax import lax
from jax.experimental import pallas as pl
from jax.experimental.pallas import tpu as pltpu
from jax.experimental.pallas import tpu_sc as plsc

N = 10000
E = 320000
F = 128
H = 20
L = 16

NP = 10240
FP = 32
NC = 2
NS = 16
NW = NC * NS
BB = 128
TOTB = (E + BB - 1) // BB
NB0 = 107
NB1 = 50
TOTB_PAD = NS * (NB0 + NB1)
NBMAX = max(NB0, NB1)
EP = (TOTB_PAD + NBMAX) * BB
RPT = NP // NS
NBUF = 8
PLA = 4
DLAG = 4

_mesh = plsc.VectorSubcoreMesh(core_axis_name="c", subcore_axis_name="s")


@functools.partial(
    pl.kernel,
    out_type=jax.ShapeDtypeStruct((NC, NP, FP), jnp.float32),
    mesh=_mesh,
    compiler_params=pltpu.CompilerParams(use_tc_tiling_on_sc=False),
    scratch_types=[
        pltpu.VMEM((NBMAX, BB), jnp.int32),
        pltpu.VMEM((NBMAX, BB), jnp.int32),
        pltpu.VMEM((NBUF, BB, FP), jnp.float32),
        pltpu.VMEM_SHARED((NP, FP), jnp.float32),
        pltpu.SemaphoreType.DMA,
        pltpu.SemaphoreType.DMA,
    ],
)
def _sc_segment_sum(y_hbm, src_hbm, dst_hbm, zeros_hbm, out_hbm,
                    src_v, dst_v, rows_v, acc, gsem, ssem):
    c = lax.axis_index("c")
    s = lax.axis_index("s")
    nb = lax.select(c == 0, NB0, NB1)
    base = c * (NS * NB0) + s * nb

    pltpu.sync_copy(zeros_hbm, acc.at[pl.ds(s * RPT, RPT)])
    pltpu.sync_copy(src_hbm.at[pl.ds(base, NBMAX)], src_v)
    pltpu.sync_copy(dst_hbm.at[pl.ds(base, NBMAX)], dst_v)
    plsc.subcore_barrier()

    for b in range(PLA):
        pltpu.async_copy(y_hbm.at[src_v.at[b]], rows_v.at[b], gsem)

    def body(j, carry):
        buf = lax.rem(j, NBUF)
        pltpu.make_async_copy(y_hbm.at[src_v.at[j]], rows_v.at[buf], gsem).wait()
        pltpu.async_copy(rows_v.at[buf], acc.at[dst_v.at[j]], ssem, add=True)

        @pl.when(j >= DLAG)
        def _drain():
            ob = lax.rem(j - DLAG, NBUF)
            pltpu.make_async_copy(rows_v.at[ob], acc.at[dst_v.at[j - DLAG]],
                                  ssem).wait()

        @pl.when(j + PLA < nb)
        def _prefetch():
            gb = lax.rem(j + PLA, NBUF)
            pltpu.async_copy(y_hbm.at[src_v.at[j + PLA]], rows_v.at[gb], gsem)

        return carry

    lax.fori_loop(0, nb, body, 0)

    def drain_body(j, carry):
        pltpu.make_async_copy(rows_v.at[lax.rem(j, NBUF)],
                              acc.at[dst_v.at[j]], ssem).wait()
        return carry

    lax.fori_loop(nb - DLAG, nb, drain_body, 0)

    plsc.subcore_barrier()
    pltpu.sync_copy(acc.at[pl.ds(s * RPT, RPT)],
                    out_hbm.at[c, pl.ds(s * RPT, RPT)])


BM = 1024


def _mm2_body(x_ref, wa_ref, wb_ref, ya_ref, yb_ref):
    xv = x_ref[...]
    ya_ref[...] = jnp.dot(xv, wa_ref[...], preferred_element_type=jnp.float32)
    yb_ref[...] = jnp.dot(xv, wb_ref[...], preferred_element_type=jnp.float32)


def _tc_mm2(xp, wa, wb):
    k = xp.shape[1]
    return pl.pallas_call(
        _mm2_body,
        grid=(NP // BM,),
        in_specs=[
            pl.BlockSpec((BM, k), lambda i: (i, 0)),
            pl.BlockSpec((k, FP), lambda i: (0, 0)),
            pl.BlockSpec((k, FP), lambda i: (0, 0)),
        ],
        out_specs=[
            pl.BlockSpec((BM, FP), lambda i: (i, 0)),
            pl.BlockSpec((BM, FP), lambda i: (i, 0)),
        ],
        out_shape=[
            jax.ShapeDtypeStruct((NP, FP), jnp.float32),
            jax.ShapeDtypeStruct((NP, FP), jnp.float32),
        ],
    )(xp, wa, wb)


def _layer_body(s_ref, r_ref, b_ref, wa_ref, wb_ref, emb_ref, ya_ref, yb_ref):
    e = jnp.maximum(s_ref[0] + s_ref[1] + r_ref[...] + b_ref[...], 0.0)
    emb_ref[...] = e
    ya_ref[...] = jnp.dot(e, wa_ref[...], preferred_element_type=jnp.float32)
    yb_ref[...] = jnp.dot(e, wb_ref[...], preferred_element_type=jnp.float32)


def _tc_layer(sparts, r, b, wa, wb):
    return pl.pallas_call(
        _layer_body,
        grid=(NP // BM,),
        in_specs=[
            pl.BlockSpec((NC, BM, FP), lambda i: (0, i, 0)),
            pl.BlockSpec((BM, FP), lambda i: (i, 0)),
            pl.BlockSpec((1, FP), lambda i: (0, 0)),
            pl.BlockSpec((FP, FP), lambda i: (0, 0)),
            pl.BlockSpec((FP, FP), lambda i: (0, 0)),
        ],
        out_specs=[
            pl.BlockSpec((BM, FP), lambda i: (i, 0)),
            pl.BlockSpec((BM, FP), lambda i: (i, 0)),
            pl.BlockSpec((BM, FP), lambda i: (i, 0)),
        ],
        out_shape=[
            jax.ShapeDtypeStruct((NP, FP), jnp.float32),
            jax.ShapeDtypeStruct((NP, FP), jnp.float32),
            jax.ShapeDtypeStruct((NP, FP), jnp.float32),
        ],
    )(sparts, r, b, wa, wb)


def _final_body(s_ref, r_ref, b_ref, e1_ref, e2_ref, w1_ref, w2_ref, w3_ref,
                bl_ref, out_ref):
    e3 = jnp.maximum(s_ref[0] + s_ref[1] + r_ref[...] + b_ref[...], 0.0)
    acc = jnp.dot(e1_ref[...], w1_ref[...], preferred_element_type=jnp.float32)
    acc += jnp.dot(e2_ref[...], w2_ref[...], preferred_element_type=jnp.float32)
    acc += jnp.dot(e3, w3_ref[...], preferred_element_type=jnp.float32)
    out_ref[...] = acc + bl_ref[...]


def _tc_final(sparts, r, b, e1, e2, w1, w2, w3, bl):
    return pl.pallas_call(
        _final_body,
        grid=(NP // BM,),
        in_specs=[
            pl.BlockSpec((NC, BM, FP), lambda i: (0, i, 0)),
            pl.BlockSpec((BM, FP), lambda i: (i, 0)),
            pl.BlockSpec((1, FP), lambda i: (0, 0)),
            pl.BlockSpec((BM, FP), lambda i: (i, 0)),
            pl.BlockSpec((BM, FP), lambda i: (i, 0)),
            pl.BlockSpec((FP, L), lambda i: (0, 0)),
            pl.BlockSpec((FP, L), lambda i: (0, 0)),
            pl.BlockSpec((FP, L), lambda i: (0, 0)),
            pl.BlockSpec((1, L), lambda i: (0, 0)),
        ],
        out_specs=pl.BlockSpec((BM, L), lambda i: (i, 0)),
        out_shape=jax.ShapeDtypeStruct((NP, L), jnp.float32),
    )(sparts, r, b, e1, e2, w1, w2, w3, bl)


def _padw(wt, rows, cols):
    out = jnp.zeros((rows, cols), jnp.float32)
    return out.at[: wt.shape[0], : wt.shape[1]].set(wt)


def kernel(x, edge_index, W_rel1, b_rel1, W_root1, W_rel2, b_rel2, W_root2,
           W_rel3, b_rel3, W_root3, W_lin, b_lin):
    src = edge_index[0].astype(jnp.int32)
    dst = edge_index[1].astype(jnp.int32)
    pad = EP - E
    srcp = jnp.concatenate([src, jnp.zeros((pad,), jnp.int32)]).reshape(-1, BB)
    dstp = jnp.concatenate([dst, jnp.full((pad,), NP - 1, jnp.int32)]).reshape(-1, BB)
    xp = jnp.pad(x, ((0, NP - N), (0, 0)))
    zeros_tile = jnp.zeros((RPT, FP), jnp.float32)

    wr1 = _padw(W_rel1.T, F, FP)
    wt1 = _padw(W_root1.T, F, FP)
    wr2 = _padw(W_rel2.T, FP, FP)
    wt2 = _padw(W_root2.T, FP, FP)
    wr3 = _padw(W_rel3.T, FP, FP)
    wt3 = _padw(W_root3.T, FP, FP)
    b1 = _padw(b_rel1[None, :], 1, FP)
    b2 = _padw(b_rel2[None, :], 1, FP)
    b3 = _padw(b_rel3[None, :], 1, FP)
    wl1 = _padw(W_lin[:, 0 * H:1 * H].T, FP, L)
    wl2 = _padw(W_lin[:, 1 * H:2 * H].T, FP, L)
    wl3 = _padw(W_lin[:, 2 * H:3 * H].T, FP, L)
    bl = b_lin[None, :]

    y1, r1 = _tc_mm2(xp, wr1, wt1)
    s1 = _sc_segment_sum(y1, srcp, dstp, zeros_tile)
    emb1, y2, r2 = _tc_layer(s1, r1, b1, wr2, wt2)
    s2 = _sc_segment_sum(y2, srcp, dstp, zeros_tile)
    emb2, y3, r3 = _tc_layer(s2, r2, b2, wr3, wt3)
    s3 = _sc_segment_sum(y3, srcp, dstp, zeros_tile)
    out = _tc_final(s3, r3, b3, emb1, emb2, wl1, wl2, wl3, bl)
    return out[:N]


# speedup vs baseline: 22.1852x; 1.2178x over previous
import functools

import jax
import jax.numpy as jnp
from jax import lax
from jax.experimental import pallas as pl
from jax.experimental.pallas import tpu as pltpu
from jax.experimental.pallas import tpu_sc as plsc

N = 10000
E = 320000
F = 128
H = 20
L = 16

NP = 10240
FP = 32
NC = 2
NS = 16
NW = NC * NS
BB = 128
TOTB = E // BB
NB0 = 106
NB1 = 51
NB1L = TOTB - NS * NB0 - (NS - 1) * NB1
NBMAX = max(NB0, NB1)
RPT = NP // NS
NBUF = 16
PLA = 8
DLAG = 8

_mesh = plsc.VectorSubcoreMesh(core_axis_name="c", subcore_axis_name="s")


@functools.partial(
    pl.kernel,
    out_type=jax.ShapeDtypeStruct((NC, NP, FP), jnp.float32),
    mesh=_mesh,
    compiler_params=pltpu.CompilerParams(use_tc_tiling_on_sc=False),
    scratch_types=[
        pltpu.VMEM((NBMAX, BB), jnp.int32),
        pltpu.VMEM((NBMAX, BB), jnp.int32),
        pltpu.VMEM((NBUF, BB, FP), jnp.float32),
        pltpu.VMEM_SHARED((NP, FP), jnp.float32),
        pltpu.SemaphoreType.DMA,
        pltpu.SemaphoreType.DMA,
    ],
)
def _sc_segment_sum(y_hbm, src_hbm, dst_hbm, zeros_hbm, out_hbm,
                    src_v, dst_v, rows_v, acc, gsem, ssem):
    c = lax.axis_index("c")
    s = lax.axis_index("s")
    nb = lax.select(c == 0, NB0, lax.select(s == NS - 1, NB1L, NB1))
    base = lax.select(c == 0, s * NB0, NS * NB0 + s * NB1)

    pltpu.async_copy(zeros_hbm.at[pl.ds(s * RPT, RPT)],
                     acc.at[pl.ds(s * RPT, RPT)], ssem)

    @pl.when(c == 0)
    def _stage0():
        pltpu.async_copy(src_hbm.at[pl.ds(base, NB0)], src_v.at[pl.ds(0, NB0)], gsem)
        pltpu.async_copy(dst_hbm.at[pl.ds(base, NB0)], dst_v.at[pl.ds(0, NB0)], gsem)

    @pl.when(jnp.logical_and(c == 1, s < NS - 1))
    def _stage1():
        pltpu.async_copy(src_hbm.at[pl.ds(base, NB1)], src_v.at[pl.ds(0, NB1)], gsem)
        pltpu.async_copy(dst_hbm.at[pl.ds(base, NB1)], dst_v.at[pl.ds(0, NB1)], gsem)

    @pl.when(jnp.logical_and(c == 1, s == NS - 1))
    def _stage1l():
        pltpu.async_copy(src_hbm.at[pl.ds(base, NB1L)], src_v.at[pl.ds(0, NB1L)], gsem)
        pltpu.async_copy(dst_hbm.at[pl.ds(base, NB1L)], dst_v.at[pl.ds(0, NB1L)], gsem)

    pltpu.make_async_copy(zeros_hbm.at[pl.ds(s * RPT, RPT)],
                          acc.at[pl.ds(s * RPT, RPT)], ssem).wait()

    @pl.when(c == 0)
    def _wait0():
        pltpu.make_async_copy(src_hbm.at[pl.ds(base, NB0)],
                              src_v.at[pl.ds(0, NB0)], gsem).wait()
        pltpu.make_async_copy(dst_hbm.at[pl.ds(base, NB0)],
                              dst_v.at[pl.ds(0, NB0)], gsem).wait()

    @pl.when(jnp.logical_and(c == 1, s < NS - 1))
    def _wait1():
        pltpu.make_async_copy(src_hbm.at[pl.ds(base, NB1)],
                              src_v.at[pl.ds(0, NB1)], gsem).wait()
        pltpu.make_async_copy(dst_hbm.at[pl.ds(base, NB1)],
                              dst_v.at[pl.ds(0, NB1)], gsem).wait()

    @pl.when(jnp.logical_and(c == 1, s == NS - 1))
    def _wait1l():
        pltpu.make_async_copy(src_hbm.at[pl.ds(base, NB1L)],
                              src_v.at[pl.ds(0, NB1L)], gsem).wait()
        pltpu.make_async_copy(dst_hbm.at[pl.ds(base, NB1L)],
                              dst_v.at[pl.ds(0, NB1L)], gsem).wait()

    plsc.subcore_barrier()

    for b in range(PLA):
        pltpu.async_copy(y_hbm.at[src_v.at[b]], rows_v.at[b], gsem)

    def body(j, carry):
        buf = lax.rem(j, NBUF)
        pltpu.make_async_copy(y_hbm.at[src_v.at[j]], rows_v.at[buf], gsem).wait()
        pltpu.async_copy(rows_v.at[buf], acc.at[dst_v.at[j]], ssem, add=True)

        @pl.when(j >= DLAG)
        def _drain():
            ob = lax.rem(j - DLAG, NBUF)
            pltpu.make_async_copy(rows_v.at[ob], acc.at[dst_v.at[j - DLAG]],
                                  ssem).wait()

        @pl.when(j + PLA < nb)
        def _prefetch():
            gb = lax.rem(j + PLA, NBUF)
            pltpu.async_copy(y_hbm.at[src_v.at[j + PLA]], rows_v.at[gb], gsem)

        return carry

    lax.fori_loop(0, nb, body, 0)

    def drain_body(j, carry):
        pltpu.make_async_copy(rows_v.at[lax.rem(j, NBUF)],
                              acc.at[dst_v.at[j]], ssem).wait()
        return carry

    lax.fori_loop(nb - DLAG, nb, drain_body, 0)

    plsc.subcore_barrier()
    pltpu.sync_copy(acc.at[pl.ds(s * RPT, RPT)],
                    out_hbm.at[c, pl.ds(s * RPT, RPT)])


BM = 1024


def _mm2_body(x_ref, wa_ref, wb_ref, ya_ref, yb_ref):
    xv = x_ref[...]
    ya_ref[...] = jnp.dot(xv, wa_ref[...], preferred_element_type=jnp.float32)
    yb_ref[...] = jnp.dot(xv, wb_ref[...], preferred_element_type=jnp.float32)


BM1 = 1000


def _tc_mm2(xp, wa, wb):
    k = xp.shape[1]
    return pl.pallas_call(
        _mm2_body,
        grid=(N // BM1,),
        in_specs=[
            pl.BlockSpec((BM1, k), lambda i: (i, 0)),
            pl.BlockSpec((k, FP), lambda i: (0, 0)),
            pl.BlockSpec((k, FP), lambda i: (0, 0)),
        ],
        out_specs=[
            pl.BlockSpec((BM1, FP), lambda i: (i, 0)),
            pl.BlockSpec((BM1, FP), lambda i: (i, 0)),
        ],
        out_shape=[
            jax.ShapeDtypeStruct((NP, FP), jnp.float32),
            jax.ShapeDtypeStruct((NP, FP), jnp.float32),
        ],
    )(xp, wa, wb)


def _layer_body(s_ref, r_ref, b_ref, wa_ref, wb_ref, emb_ref, ya_ref, yb_ref):
    e = jnp.maximum(s_ref[0] + s_ref[1] + r_ref[...] + b_ref[...], 0.0)
    emb_ref[...] = e
    ya_ref[...] = jnp.dot(e, wa_ref[...], preferred_element_type=jnp.float32)
    yb_ref[...] = jnp.dot(e, wb_ref[...], preferred_element_type=jnp.float32)


def _tc_layer(sparts, r, b, wa, wb):
    return pl.pallas_call(
        _layer_body,
        grid=(NP // BM,),
        in_specs=[
            pl.BlockSpec((NC, BM, FP), lambda i: (0, i, 0)),
            pl.BlockSpec((BM, FP), lambda i: (i, 0)),
            pl.BlockSpec((1, FP), lambda i: (0, 0)),
            pl.BlockSpec((FP, FP), lambda i: (0, 0)),
            pl.BlockSpec((FP, FP), lambda i: (0, 0)),
        ],
        out_specs=[
            pl.BlockSpec((BM, FP), lambda i: (i, 0)),
            pl.BlockSpec((BM, FP), lambda i: (i, 0)),
            pl.BlockSpec((BM, FP), lambda i: (i, 0)),
        ],
        out_shape=[
            jax.ShapeDtypeStruct((NP, FP), jnp.float32),
            jax.ShapeDtypeStruct((NP, FP), jnp.float32),
            jax.ShapeDtypeStruct((NP, FP), jnp.float32),
        ],
    )(sparts, r, b, wa, wb)


def _final_body(s_ref, r_ref, b_ref, e1_ref, e2_ref, w1_ref, w2_ref, w3_ref,
                bl_ref, out_ref):
    e3 = jnp.maximum(s_ref[0] + s_ref[1] + r_ref[...] + b_ref[...], 0.0)
    acc = jnp.dot(e1_ref[...], w1_ref[...], preferred_element_type=jnp.float32)
    acc += jnp.dot(e2_ref[...], w2_ref[...], preferred_element_type=jnp.float32)
    acc += jnp.dot(e3, w3_ref[...], preferred_element_type=jnp.float32)
    out_ref[...] = acc + bl_ref[...]


def _tc_final(sparts, r, b, e1, e2, w1, w2, w3, bl):
    return pl.pallas_call(
        _final_body,
        grid=(NP // BM,),
        in_specs=[
            pl.BlockSpec((NC, BM, FP), lambda i: (0, i, 0)),
            pl.BlockSpec((BM, FP), lambda i: (i, 0)),
            pl.BlockSpec((1, FP), lambda i: (0, 0)),
            pl.BlockSpec((BM, FP), lambda i: (i, 0)),
            pl.BlockSpec((BM, FP), lambda i: (i, 0)),
            pl.BlockSpec((FP, L), lambda i: (0, 0)),
            pl.BlockSpec((FP, L), lambda i: (0, 0)),
            pl.BlockSpec((FP, L), lambda i: (0, 0)),
            pl.BlockSpec((1, L), lambda i: (0, 0)),
        ],
        out_specs=pl.BlockSpec((BM, L), lambda i: (i, 0)),
        out_shape=jax.ShapeDtypeStruct((NP, L), jnp.float32),
    )(sparts, r, b, e1, e2, w1, w2, w3, bl)


def _padw(wt, rows, cols):
    out = jnp.zeros((rows, cols), jnp.float32)
    return out.at[: wt.shape[0], : wt.shape[1]].set(wt)


def kernel(x, edge_index, W_rel1, b_rel1, W_root1, W_rel2, b_rel2, W_root2,
           W_rel3, b_rel3, W_root3, W_lin, b_lin):
    srcp = edge_index[0].astype(jnp.int32).reshape(TOTB, BB)
    dstp = edge_index[1].astype(jnp.int32).reshape(TOTB, BB)
    zeros_full = jnp.zeros((NP, FP), jnp.float32)

    wr1 = _padw(W_rel1.T, F, FP)
    wt1 = _padw(W_root1.T, F, FP)
    wr2 = _padw(W_rel2.T, FP, FP)
    wt2 = _padw(W_root2.T, FP, FP)
    wr3 = _padw(W_rel3.T, FP, FP)
    wt3 = _padw(W_root3.T, FP, FP)
    b1 = _padw(b_rel1[None, :], 1, FP)
    b2 = _padw(b_rel2[None, :], 1, FP)
    b3 = _padw(b_rel3[None, :], 1, FP)
    wl1 = _padw(W_lin[:, 0 * H:1 * H].T, FP, L)
    wl2 = _padw(W_lin[:, 1 * H:2 * H].T, FP, L)
    wl3 = _padw(W_lin[:, 2 * H:3 * H].T, FP, L)
    bl = b_lin[None, :]

    y1, r1 = _tc_mm2(x, wr1, wt1)
    s1 = _sc_segment_sum(y1, srcp, dstp, zeros_full)
    emb1, y2, r2 = _tc_layer(s1, r1, b1, wr2, wt2)
    s2 = _sc_segment_sum(y2, srcp, dstp, zeros_full)
    emb2, y3, r3 = _tc_layer(s2, r2, b2, wr3, wt3)
    s3 = _sc_segment_sum(y3, srcp, dstp, zeros_full)
    out = _tc_final(s3, r3, b3, emb1, emb2, wl1, wl2, wl3, bl)
    return out[:N]


# speedup vs baseline: 24.3275x; 1.0966x over previous
import functools

import jax
import jax.numpy as jnp
from jax import lax
from jax.experimental import pallas as pl
from jax.experimental.pallas import tpu as pltpu
from jax.experimental.pallas import tpu_sc as plsc

N = 10000
E = 320000
F = 128
H = 20
L = 16

NP = 10240
FP = 32
NC = 2
NS = 16
NW = NC * NS
BB = 128
TOTB = E // BB
NB0 = 86
NB1 = 71
NB1L = TOTB - NS * NB0 - (NS - 1) * NB1
NBMAX = max(NB0, NB1)
RPT = NP // NS
NBUF = 20
PLA = 10
DLAG = 10

_mesh = plsc.VectorSubcoreMesh(core_axis_name="c", subcore_axis_name="s")


@functools.partial(
    pl.kernel,
    out_type=jax.ShapeDtypeStruct((NC, NP, FP), jnp.float32),
    mesh=_mesh,
    compiler_params=pltpu.CompilerParams(use_tc_tiling_on_sc=False),
    scratch_types=[
        pltpu.VMEM((NBMAX, BB), jnp.int32),
        pltpu.VMEM((NBMAX, BB), jnp.int32),
        pltpu.VMEM((NBUF, BB, FP), jnp.float32),
        pltpu.VMEM_SHARED((NP, FP), jnp.float32),
        pltpu.SemaphoreType.DMA,
        pltpu.SemaphoreType.DMA,
    ],
)
def _sc_segment_sum(y_hbm, src_hbm, dst_hbm, zeros_hbm, out_hbm,
                    src_v, dst_v, rows_v, acc, gsem, ssem):
    c = lax.axis_index("c")
    s = lax.axis_index("s")
    nb = lax.select(c == 0, NB0, lax.select(s == NS - 1, NB1L, NB1))
    base = lax.select(c == 0, s * NB0, NS * NB0 + s * NB1)

    pltpu.async_copy(zeros_hbm.at[pl.ds(s * RPT, RPT)],
                     acc.at[pl.ds(s * RPT, RPT)], ssem)

    @pl.when(c == 0)
    def _stage0():
        pltpu.async_copy(src_hbm.at[pl.ds(base, NB0)], src_v.at[pl.ds(0, NB0)], gsem)
        pltpu.async_copy(dst_hbm.at[pl.ds(base, NB0)], dst_v.at[pl.ds(0, NB0)], gsem)

    @pl.when(jnp.logical_and(c == 1, s < NS - 1))
    def _stage1():
        pltpu.async_copy(src_hbm.at[pl.ds(base, NB1)], src_v.at[pl.ds(0, NB1)], gsem)
        pltpu.async_copy(dst_hbm.at[pl.ds(base, NB1)], dst_v.at[pl.ds(0, NB1)], gsem)

    @pl.when(jnp.logical_and(c == 1, s == NS - 1))
    def _stage1l():
        pltpu.async_copy(src_hbm.at[pl.ds(base, NB1L)], src_v.at[pl.ds(0, NB1L)], gsem)
        pltpu.async_copy(dst_hbm.at[pl.ds(base, NB1L)], dst_v.at[pl.ds(0, NB1L)], gsem)

    pltpu.make_async_copy(zeros_hbm.at[pl.ds(s * RPT, RPT)],
                          acc.at[pl.ds(s * RPT, RPT)], ssem).wait()

    @pl.when(c == 0)
    def _wait0():
        pltpu.make_async_copy(src_hbm.at[pl.ds(base, NB0)],
                              src_v.at[pl.ds(0, NB0)], gsem).wait()
        pltpu.make_async_copy(dst_hbm.at[pl.ds(base, NB0)],
                              dst_v.at[pl.ds(0, NB0)], gsem).wait()

    @pl.when(jnp.logical_and(c == 1, s < NS - 1))
    def _wait1():
        pltpu.make_async_copy(src_hbm.at[pl.ds(base, NB1)],
                              src_v.at[pl.ds(0, NB1)], gsem).wait()
        pltpu.make_async_copy(dst_hbm.at[pl.ds(base, NB1)],
                              dst_v.at[pl.ds(0, NB1)], gsem).wait()

    @pl.when(jnp.logical_and(c == 1, s == NS - 1))
    def _wait1l():
        pltpu.make_async_copy(src_hbm.at[pl.ds(base, NB1L)],
                              src_v.at[pl.ds(0, NB1L)], gsem).wait()
        pltpu.make_async_copy(dst_hbm.at[pl.ds(base, NB1L)],
                              dst_v.at[pl.ds(0, NB1L)], gsem).wait()

    plsc.subcore_barrier()

    for b in range(PLA):
        pltpu.async_copy(y_hbm.at[src_v.at[b]], rows_v.at[b], gsem)

    def body(j, carry):
        buf = lax.rem(j, NBUF)
        pltpu.make_async_copy(y_hbm.at[src_v.at[j]], rows_v.at[buf], gsem).wait()
        pltpu.async_copy(rows_v.at[buf], acc.at[dst_v.at[j]], ssem, add=True)

        @pl.when(j >= DLAG)
        def _drain():
            ob = lax.rem(j - DLAG, NBUF)
            pltpu.make_async_copy(rows_v.at[ob], acc.at[dst_v.at[j - DLAG]],
                                  ssem).wait()

        @pl.when(j + PLA < nb)
        def _prefetch():
            gb = lax.rem(j + PLA, NBUF)
            pltpu.async_copy(y_hbm.at[src_v.at[j + PLA]], rows_v.at[gb], gsem)

        return carry

    lax.fori_loop(0, nb, body, 0)

    def drain_body(j, carry):
        pltpu.make_async_copy(rows_v.at[lax.rem(j, NBUF)],
                              acc.at[dst_v.at[j]], ssem).wait()
        return carry

    lax.fori_loop(nb - DLAG, nb, drain_body, 0)

    plsc.subcore_barrier()
    pltpu.sync_copy(acc.at[pl.ds(s * RPT, RPT)],
                    out_hbm.at[c, pl.ds(s * RPT, RPT)])


BM = 2560


def _mm2_body(x_ref, wa_ref, wb_ref, ya_ref, yb_ref):
    xv = x_ref[...]
    ya_ref[...] = jnp.dot(xv, wa_ref[...], preferred_element_type=jnp.float32)
    yb_ref[...] = jnp.dot(xv, wb_ref[...], preferred_element_type=jnp.float32)


BM1 = 1000


def _tc_mm2(xp, wa, wb):
    k = xp.shape[1]
    return pl.pallas_call(
        _mm2_body,
        grid=(N // BM1,),
        in_specs=[
            pl.BlockSpec((BM1, k), lambda i: (i, 0)),
            pl.BlockSpec((k, FP), lambda i: (0, 0)),
            pl.BlockSpec((k, FP), lambda i: (0, 0)),
        ],
        out_specs=[
            pl.BlockSpec((BM1, FP), lambda i: (i, 0)),
            pl.BlockSpec((BM1, FP), lambda i: (i, 0)),
        ],
        out_shape=[
            jax.ShapeDtypeStruct((NP, FP), jnp.float32),
            jax.ShapeDtypeStruct((NP, FP), jnp.float32),
        ],
    )(xp, wa, wb)


def _layer_body(s_ref, r_ref, b_ref, wa_ref, wb_ref, emb_ref, ya_ref, yb_ref):
    e = jnp.maximum(s_ref[0] + s_ref[1] + r_ref[...] + b_ref[...], 0.0)
    emb_ref[...] = e
    ya_ref[...] = jnp.dot(e, wa_ref[...], preferred_element_type=jnp.float32)
    yb_ref[...] = jnp.dot(e, wb_ref[...], preferred_element_type=jnp.float32)


def _tc_layer(sparts, r, b, wa, wb):
    return pl.pallas_call(
        _layer_body,
        grid=(NP // BM,),
        in_specs=[
            pl.BlockSpec((NC, BM, FP), lambda i: (0, i, 0)),
            pl.BlockSpec((BM, FP), lambda i: (i, 0)),
            pl.BlockSpec((1, FP), lambda i: (0, 0)),
            pl.BlockSpec((FP, FP), lambda i: (0, 0)),
            pl.BlockSpec((FP, FP), lambda i: (0, 0)),
        ],
        out_specs=[
            pl.BlockSpec((BM, FP), lambda i: (i, 0)),
            pl.BlockSpec((BM, FP), lambda i: (i, 0)),
            pl.BlockSpec((BM, FP), lambda i: (i, 0)),
        ],
        out_shape=[
            jax.ShapeDtypeStruct((NP, FP), jnp.float32),
            jax.ShapeDtypeStruct((NP, FP), jnp.float32),
            jax.ShapeDtypeStruct((NP, FP), jnp.float32),
        ],
    )(sparts, r, b, wa, wb)


def _final_body(s_ref, r_ref, b_ref, e1_ref, e2_ref, w1_ref, w2_ref, w3_ref,
                bl_ref, out_ref):
    e3 = jnp.maximum(s_ref[0] + s_ref[1] + r_ref[...] + b_ref[...], 0.0)
    acc = jnp.dot(e1_ref[...], w1_ref[...], preferred_element_type=jnp.float32)
    acc += jnp.dot(e2_ref[...], w2_ref[...], preferred_element_type=jnp.float32)
    acc += jnp.dot(e3, w3_ref[...], preferred_element_type=jnp.float32)
    out_ref[...] = acc + bl_ref[...]


def _tc_final(sparts, r, b, e1, e2, w1, w2, w3, bl):
    return pl.pallas_call(
        _final_body,
        grid=(NP // BM,),
        in_specs=[
            pl.BlockSpec((NC, BM, FP), lambda i: (0, i, 0)),
            pl.BlockSpec((BM, FP), lambda i: (i, 0)),
            pl.BlockSpec((1, FP), lambda i: (0, 0)),
            pl.BlockSpec((BM, FP), lambda i: (i, 0)),
            pl.BlockSpec((BM, FP), lambda i: (i, 0)),
            pl.BlockSpec((FP, L), lambda i: (0, 0)),
            pl.BlockSpec((FP, L), lambda i: (0, 0)),
            pl.BlockSpec((FP, L), lambda i: (0, 0)),
            pl.BlockSpec((1, L), lambda i: (0, 0)),
        ],
        out_specs=pl.BlockSpec((BM, L), lambda i: (i, 0)),
        out_shape=jax.ShapeDtypeStruct((NP, L), jnp.float32),
    )(sparts, r, b, e1, e2, w1, w2, w3, bl)


def _padw(wt, rows, cols):
    out = jnp.zeros((rows, cols), jnp.float32)
    return out.at[: wt.shape[0], : wt.shape[1]].set(wt)


def kernel(x, edge_index, W_rel1, b_rel1, W_root1, W_rel2, b_rel2, W_root2,
           W_rel3, b_rel3, W_root3, W_lin, b_lin):
    srcp = edge_index[0].astype(jnp.int32).reshape(TOTB, BB)
    dstp = edge_index[1].astype(jnp.int32).reshape(TOTB, BB)
    zeros_full = jnp.zeros((NP, FP), jnp.float32)

    wr1 = _padw(W_rel1.T, F, FP)
    wt1 = _padw(W_root1.T, F, FP)
    wr2 = _padw(W_rel2.T, FP, FP)
    wt2 = _padw(W_root2.T, FP, FP)
    wr3 = _padw(W_rel3.T, FP, FP)
    wt3 = _padw(W_root3.T, FP, FP)
    b1 = _padw(b_rel1[None, :], 1, FP)
    b2 = _padw(b_rel2[None, :], 1, FP)
    b3 = _padw(b_rel3[None, :], 1, FP)
    wl1 = _padw(W_lin[:, 0 * H:1 * H].T, FP, L)
    wl2 = _padw(W_lin[:, 1 * H:2 * H].T, FP, L)
    wl3 = _padw(W_lin[:, 2 * H:3 * H].T, FP, L)
    bl = b_lin[None, :]

    y1, r1 = _tc_mm2(x, wr1, wt1)
    s1 = _sc_segment_sum(y1, srcp, dstp, zeros_full)
    emb1, y2, r2 = _tc_layer(s1, r1, b1, wr2, wt2)
    s2 = _sc_segment_sum(y2, srcp, dstp, zeros_full)
    emb2, y3, r3 = _tc_layer(s2, r2, b2, wr3, wt3)
    s3 = _sc_segment_sum(y3, srcp, dstp, zeros_full)
    out = _tc_final(s3, r3, b3, emb1, emb2, wl1, wl2, wl3, bl)
    return out[:N]
